# speedup vs baseline: 13.5529x; 13.5529x over previous
import functools

import jax
import jax.numpy as jnp
from jax import lax
from jax.experimental import pallas as pl
from jax.experimental.pallas import tpu as pltpu
from jax.experimental.pallas import tpu_sc as plsc

B = 1024
D = 4096
NC = 2
NS = 16
NW = NC * NS
BPW = B // NW
CH = 8
NCH = BPW // CH

_mesh = plsc.VectorSubcoreMesh(core_axis_name="c", subcore_axis_name="s")


@functools.partial(
    pl.kernel,
    mesh=_mesh,
    out_type=jax.ShapeDtypeStruct((B, D), jnp.float32),
    scratch_types=[
        pltpu.VMEM((BPW,), jnp.int32),
        pltpu.VMEM((CH, D), jnp.float32),
        pltpu.VMEM((CH, D), jnp.float32),
        pltpu.SemaphoreType.DMA,
        pltpu.SemaphoreType.DMA,
    ],
)
def _gather_rows(idx_hbm, table_hbm, out_hbm, idx_v, buf0, buf1, sem0, sem1):
    wid = lax.axis_index("s") * NC + lax.axis_index("c")
    base = wid * BPW
    pltpu.sync_copy(idx_hbm.at[pl.ds(base, BPW)], idx_v)
    bufs = (buf0, buf1)
    sems = (sem0, sem1)
    pending = pltpu.async_copy(
        table_hbm.at[idx_v.at[pl.ds(0, CH)]], bufs[0], sems[0])
    for g in range(NCH):
        nxt = None
        if g + 1 < NCH:
            nxt = pltpu.async_copy(
                table_hbm.at[idx_v.at[pl.ds((g + 1) * CH, CH)]],
                bufs[(g + 1) % 2], sems[(g + 1) % 2])
        pending.wait()
        pltpu.sync_copy(bufs[g % 2], out_hbm.at[pl.ds(base + g * CH, CH)])
        pending = nxt


def kernel(x, table):
    idx = x[:, 0]
    out = _gather_rows(idx, table)
    return (out, out)


# speedup vs baseline: 16.3690x; 1.2078x over previous
import functools

import jax
import jax.numpy as jnp
from jax import lax
from jax.experimental import pallas as pl
from jax.experimental.pallas import tpu as pltpu
from jax.experimental.pallas import tpu_sc as plsc

B = 1024
D = 4096
NC = 2
NS = 16
NW = NC * NS
BPW = B // NW
CH = 8
NCH = BPW // CH

_mesh = plsc.VectorSubcoreMesh(core_axis_name="c", subcore_axis_name="s")


@functools.partial(
    pl.kernel,
    mesh=_mesh,
    out_type=(
        jax.ShapeDtypeStruct((B, D), jnp.float32),
        jax.ShapeDtypeStruct((B, D), jnp.float32),
    ),
    scratch_types=[
        pltpu.VMEM((BPW,), jnp.int32),
        pltpu.VMEM((CH, D), jnp.float32),
        pltpu.VMEM((CH, D), jnp.float32),
        pltpu.SemaphoreType.DMA,
        pltpu.SemaphoreType.DMA,
        pltpu.SemaphoreType.DMA,
        pltpu.SemaphoreType.DMA,
        pltpu.SemaphoreType.DMA,
        pltpu.SemaphoreType.DMA,
    ],
)
def _gather_rows(idx_hbm, table_hbm, out1_hbm, out2_hbm, idx_v,
                 buf0, buf1, g0, g1, w00, w01, w10, w11):
    wid = lax.axis_index("s") * NC + lax.axis_index("c")
    base = wid * BPW
    pltpu.sync_copy(idx_hbm.at[pl.ds(base, BPW)], idx_v)
    bufs = (buf0, buf1)
    gsems = (g0, g1)
    wsems = ((w00, w01), (w10, w11))
    pend_g = {0: pltpu.async_copy(
        table_hbm.at[idx_v.at[pl.ds(0, CH)]], bufs[0], gsems[0])}
    pend_w = {}
    for g in range(NCH):
        if g + 1 < NCH:
            if g - 1 >= 0:
                for c in pend_w[g - 1]:
                    c.wait()
            pend_g[g + 1] = pltpu.async_copy(
                table_hbm.at[idx_v.at[pl.ds((g + 1) * CH, CH)]],
                bufs[(g + 1) % 2], gsems[(g + 1) % 2])
        pend_g[g].wait()
        rows = pl.ds(base + g * CH, CH)
        pend_w[g] = (
            pltpu.async_copy(bufs[g % 2], out1_hbm.at[rows], wsems[g % 2][0]),
            pltpu.async_copy(bufs[g % 2], out2_hbm.at[rows], wsems[g % 2][1]),
        )
    for g in (NCH - 2, NCH - 1):
        for c in pend_w[g]:
            c.wait()


def kernel(x, table):
    out1, out2 = _gather_rows(x[:, 0], table)
    return (out1, out2)
